# baseline (speedup 1.0000x reference)
import functools

import jax
import jax.numpy as jnp
from jax.experimental import pallas as pl
from jax.experimental.pallas import tpu as pltpu

_BM = 512
_BK = 512


def _gcn_block_kernel(ws_ref, x_ref, w_ref, adj_ref, bias_ref, out_ref,
                      support_ref, *, bk):
    i = pl.program_id(0)
    j = pl.program_id(1)

    @pl.when(i == 0)
    def _compute_support():
        support_ref[pl.ds(j * bk, bk), :] = jnp.dot(
            x_ref[...], w_ref[...], preferred_element_type=jnp.float32)

    r = adj_ref.shape[0]
    wadj = adj_ref[0] * ws_ref[0]
    for k in range(1, r):
        wadj = wadj + adj_ref[k] * ws_ref[k]
    part = jnp.dot(wadj, support_ref[pl.ds(j * bk, bk), :],
                   preferred_element_type=jnp.float32)

    @pl.when(j == 0)
    def _init():
        out_ref[...] = part + bias_ref[...]

    @pl.when(j != 0)
    def _accum():
        out_ref[...] += part


def kernel(input, adj, weight, weight_sum, bias):
    n, d_in = input.shape
    d_out = weight.shape[1]
    r = adj.shape[0]
    gi, gj = n // _BM, n // _BK

    return pl.pallas_call(
        functools.partial(_gcn_block_kernel, bk=_BK),
        grid=(gi, gj),
        in_specs=[
            pl.BlockSpec(memory_space=pltpu.SMEM),
            pl.BlockSpec((_BK, d_in),
                         lambda i, j: (jax.lax.select(i == 0, j, 0), 0)),
            pl.BlockSpec((d_in, d_out), lambda i, j: (0, 0)),
            pl.BlockSpec((r, _BM, _BK), lambda i, j: (0, i, j)),
            pl.BlockSpec((1, d_out), lambda i, j: (0, 0)),
        ],
        out_specs=pl.BlockSpec((_BM, d_out), lambda i, j: (i, 0)),
        out_shape=jax.ShapeDtypeStruct((n, d_out), jnp.float32),
        scratch_shapes=[pltpu.VMEM((n, d_out), jnp.float32)],
    )(weight_sum, input, weight, adj, bias.reshape(1, d_out))


# speedup vs baseline: 1.2280x; 1.2280x over previous
import functools

import jax
import jax.numpy as jnp
from jax.experimental import pallas as pl
from jax.experimental.pallas import tpu as pltpu

_BM = 256


def _gcn_block_kernel(ws_ref, x_ref, w_ref, adj_ref, bias_ref, out_ref,
                      support_ref):
    i = pl.program_id(0)

    @pl.when(i == 0)
    def _compute_support():
        support_ref[...] = jnp.dot(
            x_ref[...], w_ref[...], preferred_element_type=jnp.float32)

    r = adj_ref.shape[0]
    wadj = adj_ref[0] * ws_ref[0]
    for k in range(1, r):
        wadj = wadj + adj_ref[k] * ws_ref[k]
    out_ref[...] = jnp.dot(wadj, support_ref[...],
                           preferred_element_type=jnp.float32) + bias_ref[...]


def kernel(input, adj, weight, weight_sum, bias):
    n, d_in = input.shape
    d_out = weight.shape[1]
    r = adj.shape[0]
    gi = n // _BM

    return pl.pallas_call(
        _gcn_block_kernel,
        grid=(gi,),
        in_specs=[
            pl.BlockSpec(memory_space=pltpu.SMEM),
            pl.BlockSpec((n, d_in), lambda i: (0, 0)),
            pl.BlockSpec((d_in, d_out), lambda i: (0, 0)),
            pl.BlockSpec((r, _BM, n), lambda i: (0, i, 0)),
            pl.BlockSpec((1, d_out), lambda i: (0, 0)),
        ],
        out_specs=pl.BlockSpec((_BM, d_out), lambda i: (i, 0)),
        out_shape=jax.ShapeDtypeStruct((n, d_out), jnp.float32),
        scratch_shapes=[pltpu.VMEM((n, d_out), jnp.float32)],
    )(weight_sum, input, weight, adj, bias.reshape(1, d_out))
